# Initial kernel scaffold; baseline (speedup 1.0000x reference)
import jax
import jax.numpy as jnp
from jax.experimental import pallas as pl


def kernel(nf_init, edge_index, W_self, W_msg, W_agg, b_g, W_a1, b_a1, w_a2, w_c, b_c):
    raise NotImplementedError("write your pallas kernel here")


# speedup vs baseline: 5.0485x; 5.0485x over previous
import functools

import jax
import jax.numpy as jnp
from jax import lax
from jax.experimental import pallas as pl
from jax.experimental.pallas import tpu as pltpu
from jax.experimental.pallas import tpu_sc as plsc

N = 10000
E = 320000
D = 128
H = 128
NC = 2
NS = 16
NW = NC * NS
EPT = E // NW
CH = 128
NFULL = EPT // CH
REM = EPT - NFULL * CH
NP = 10240
RPT = NP // NS

_mesh = plsc.VectorSubcoreMesh(
    core_axis_name="c", subcore_axis_name="s", num_cores=NC, num_subcores=NS)


@functools.partial(
    pl.kernel,
    out_type=[
        jax.ShapeDtypeStruct((NC, NP, D), jnp.float32),
        jax.ShapeDtypeStruct((NC, NS, NP), jnp.float32),
    ],
    mesh=_mesh,
    scratch_types=[
        pltpu.VMEM((CH,), jnp.int32),
        pltpu.VMEM((CH,), jnp.int32),
        pltpu.VMEM((CH, D), jnp.float32),
        pltpu.VMEM((REM,), jnp.int32),
        pltpu.VMEM((REM,), jnp.int32),
        pltpu.VMEM((REM, D), jnp.float32),
        pltpu.VMEM((NP,), jnp.float32),
        pltpu.VMEM_SHARED((NP, D), jnp.float32),
        pltpu.SemaphoreType.DMA,
    ],
    compiler_params=pltpu.CompilerParams(needs_layout_passes=False),
)
def _scatter_add_kernel(src_hbm, dst_hbm, nf_hbm, zeros_hbm, s_hbm, deg_hbm,
                        sidx, didx, rows, sidx_r, didx_r, rows_r, degbuf, acc, sem):
    c = lax.axis_index("c")
    s = lax.axis_index("s")
    wid = s * NC + c
    roff = s * RPT
    pltpu.sync_copy(zeros_hbm.at[pl.ds(roff, RPT), :], acc.at[pl.ds(roff, RPT), :])

    def zbody(i, _):
        degbuf[pl.ds(i * 16, 16)] = jnp.zeros((16,), jnp.float32)
        return _

    lax.fori_loop(0, NP // 16, zbody, 0)
    plsc.subcore_barrier()
    base = wid * EPT

    def handle(off, n, si, di, rbuf):
        pltpu.sync_copy(src_hbm.at[pl.ds(off, n)], si)
        pltpu.sync_copy(dst_hbm.at[pl.ds(off, n)], di)
        pltpu.async_copy(nf_hbm.at[si], rbuf, sem).wait()
        pltpu.sync_copy(rbuf, acc.at[di], add=True)

        def dbody(g, _):
            dv = di[pl.ds(g * 16, 16)]
            cnt, lastm = plsc.scan_count(dv)
            plsc.addupdate_scatter(degbuf, [dv], cnt.astype(jnp.float32),
                                   mask=lastm)
            return _

        lax.fori_loop(0, n // 16, dbody, 0)

    def body(i, _):
        handle(base + i * CH, CH, sidx, didx, rows)
        return _

    lax.fori_loop(0, NFULL, body, 0)
    handle(base + NFULL * CH, REM, sidx_r, didx_r, rows_r)
    plsc.subcore_barrier()
    pltpu.sync_copy(acc.at[pl.ds(roff, RPT), :], s_hbm.at[c, pl.ds(roff, RPT), :])
    pltpu.sync_copy(degbuf, deg_hbm.at[c, s])


_NBLK = 10
_BR = NP // _NBLK


def _dense_body(sp_ref, degs_ref, nf_ref, wself_ref, wmsg_ref, wagg_ref, bg_ref,
                wa1_ref, ba1_ref, wc_ref, bc_ref,
                p_ref, q_ref, sv_ref):
    i = pl.program_id(0)
    sp = sp_ref[...]
    s = sp[0] + sp[1]
    deg = jnp.sum(degs_ref[...], axis=(0, 1))
    t = s / jnp.clip(deg, 1.0, None)[:, None]
    hp = jax.lax.Precision.HIGHEST
    a = jnp.dot(jnp.dot(t, wmsg_ref[...], precision=hp), wagg_ref[...], precision=hp)
    ne = jax.nn.relu(jnp.dot(nf_ref[...], wself_ref[...], precision=hp) + a + bg_ref[...])
    wa1 = wa1_ref[...]
    p_ref[...] = jnp.dot(ne, wa1[:H, :], precision=hp) + ba1_ref[...]
    q_ref[...] = jnp.dot(ne, wa1[H:, :], precision=hp)

    @pl.when(i == 0)
    def _():
        sv_ref[...] = jnp.zeros_like(sv_ref)

    rowmask = i * _BR + lax.broadcasted_iota(jnp.int32, (_BR, 1), 0) < N
    sv_ref[...] += jnp.sum(jnp.where(rowmask, ne * wc_ref[...], 0.0)).reshape(1, 1)

    @pl.when(i == _NBLK - 1)
    def _():
        sv_ref[...] = sv_ref[...] / jnp.float32(N) + bc_ref[...]


_dense_call = pl.pallas_call(
    _dense_body,
    grid=(_NBLK,),
    in_specs=[
        pl.BlockSpec((NC, _BR, D), lambda i: (0, i, 0)),
        pl.BlockSpec((NC, NS, _BR), lambda i: (0, 0, i)),
        pl.BlockSpec((_BR, D), lambda i: (i, 0)),
        pl.BlockSpec((D, H), lambda i: (0, 0)),
        pl.BlockSpec((D, H), lambda i: (0, 0)),
        pl.BlockSpec((H, H), lambda i: (0, 0)),
        pl.BlockSpec((1, H), lambda i: (0, 0)),
        pl.BlockSpec((2 * H, H), lambda i: (0, 0)),
        pl.BlockSpec((1, H), lambda i: (0, 0)),
        pl.BlockSpec((1, H), lambda i: (0, 0)),
        pl.BlockSpec((1, 1), lambda i: (0, 0)),
    ],
    out_specs=[
        pl.BlockSpec((_BR, H), lambda i: (i, 0)),
        pl.BlockSpec((_BR, H), lambda i: (i, 0)),
        pl.BlockSpec((1, 1), lambda i: (0, 0)),
    ],
    out_shape=[
        jax.ShapeDtypeStruct((NP, H), jnp.float32),
        jax.ShapeDtypeStruct((NP, H), jnp.float32),
        jax.ShapeDtypeStruct((1, 1), jnp.float32),
    ],
)


@functools.partial(
    pl.kernel,
    out_type=jax.ShapeDtypeStruct((E,), jnp.float32),
    mesh=_mesh,
    scratch_types=[
        pltpu.VMEM((CH,), jnp.int32),
        pltpu.VMEM((CH,), jnp.int32),
        pltpu.VMEM((CH, H), jnp.float32),
        pltpu.VMEM((CH, H), jnp.float32),
        pltpu.VMEM((CH,), jnp.float32),
        pltpu.VMEM((REM,), jnp.int32),
        pltpu.VMEM((REM,), jnp.int32),
        pltpu.VMEM((REM, H), jnp.float32),
        pltpu.VMEM((REM, H), jnp.float32),
        pltpu.VMEM((REM,), jnp.float32),
        pltpu.VMEM((H,), jnp.float32),
        pltpu.SemaphoreType.DMA,
        pltpu.SemaphoreType.DMA,
    ],
    compiler_params=pltpu.CompilerParams(needs_layout_passes=False),
)
def _edge_logits_kernel(src_hbm, dst_hbm, p_hbm, q_hbm, w2_hbm, out_hbm,
                        sidx, didx, prow, qrow, lbuf,
                        sidx_r, didx_r, prow_r, qrow_r, lbuf_r, w2, sem1, sem2):
    c = lax.axis_index("c")
    s = lax.axis_index("s")
    wid = s * NC + c
    base = wid * EPT
    pltpu.sync_copy(w2_hbm, w2)
    w2v = [w2[pl.ds(k * 16, 16)] for k in range(H // 16)]

    lanes = lax.iota(jnp.int32, 16)

    def do_chunk(off, n, si, di, pr, qr, lb):
        pltpu.sync_copy(src_hbm.at[pl.ds(off, n)], si)
        pltpu.sync_copy(dst_hbm.at[pl.ds(off, n)], di)
        cp = pltpu.async_copy(p_hbm.at[si], pr, sem1)
        cq = pltpu.async_copy(q_hbm.at[di], qr, sem2)
        cp.wait()
        cq.wait()

        def group(g, _):
            lvec = jnp.zeros((16,), jnp.float32)
            for j in range(16):
                e = g * 16 + j
                acc = jnp.zeros((16,), jnp.float32)
                for k in range(H // 16):
                    pv = pr[e, pl.ds(k * 16, 16)]
                    qv = qr[e, pl.ds(k * 16, 16)]
                    acc = acc + jnp.maximum(pv + qv, 0.0) * w2v[k]
                lvec = jnp.where(lanes == j, jnp.sum(acc), lvec)
            lb[pl.ds(g * 16, 16)] = lvec
            return _

        lax.fori_loop(0, n // 16, group, 0)
        pltpu.sync_copy(lb, out_hbm.at[pl.ds(off, n)])

    def body(i, _):
        do_chunk(base + i * CH, CH, sidx, didx, prow, qrow, lbuf)
        return _

    lax.fori_loop(0, NFULL, body, 0)
    do_chunk(base + NFULL * CH, REM, sidx_r, didx_r, prow_r, qrow_r, lbuf_r)


_LROWS = E // 128


def _finalize_body(l_ref, g_ref, a_ref, lp_ref):
    l = l_ref[...]
    m = jnp.max(l)
    lse = jnp.log(jnp.sum(jnp.exp(l - m))) + m
    y = l + g_ref[...]
    ym = jnp.max(y)
    flat = (lax.broadcasted_iota(jnp.int32, (_LROWS, 128), 0) * 128
            + lax.broadcasted_iota(jnp.int32, (_LROWS, 128), 1))
    big = jnp.int32(2**30)
    am = jnp.min(jnp.where(y == ym, flat, big))
    lsel = jnp.max(jnp.where(flat == am, l, -jnp.inf))
    a_ref[...] = am.reshape(1, 1)
    lp_ref[...] = (lsel - lse).reshape(1, 1)


_finalize_call = pl.pallas_call(
    _finalize_body,
    out_shape=[
        jax.ShapeDtypeStruct((1, 1), jnp.int32),
        jax.ShapeDtypeStruct((1, 1), jnp.float32),
    ],
)


def kernel(nf_init, edge_index, W_self, W_msg, W_agg, b_g, W_a1, b_a1, w_a2, w_c, b_c):
    src = edge_index[0]
    dst = edge_index[1]
    zeros = jnp.zeros((NP, D), jnp.float32)

    sparts, degs = _scatter_add_kernel(src, dst, nf_init, zeros)

    p, q, sv = _dense_call(
        sparts, degs, nf_init, W_self, W_msg, W_agg, b_g.reshape(1, H),
        W_a1, b_a1.reshape(1, H), w_c.reshape(1, H),
        b_c.reshape(1, 1))

    logits = _edge_logits_kernel(src, dst, p, q, w_a2)

    gumbel = jax.random.gumbel(jax.random.key(42), (E,), jnp.float32)
    a, lp = _finalize_call(logits.reshape(_LROWS, 128), gumbel.reshape(_LROWS, 128))

    return (a[0, 0], lp[0, 0], sv[0, 0])
